# baseline (speedup 1.0000x reference)
import functools

import jax
import jax.numpy as jnp
from jax import lax
from jax.experimental import pallas as pl
from jax.experimental.pallas import tpu as pltpu
from jax.experimental.pallas import tpu_sc as plsc

NC, NS, L = 2, 16, 16
NW = NC * NS


def _worker_id():
    return lax.axis_index("s") * NC + lax.axis_index("c")


def _body(x_hbm, pos_hbm, t0_hbm, t1_hbm, h0_hbm, h1_hbm, out_hbm,
          xv, posv, fidx, i0, i1, r0, r1, sem, *, b_per_w, n_chunks):
    wid = _worker_id()
    base = wid * b_per_w
    P = b_per_w * n_chunks
    KV = P // L

    pltpu.sync_copy(x_hbm.at[pl.ds(base, b_per_w)], xv)
    pltpu.sync_copy(pos_hbm, posv)

    lane = lax.iota(jnp.int32, L)
    pat = lane % n_chunks

    def fill(k, _):
        pos = posv[pl.ds(k * L, L)]
        xi = plsc.load_gather(xv, [pos])
        fidx[pl.ds(k * L, L)] = xi * n_chunks + pat
        return _
    lax.fori_loop(0, KV, fill, None)

    c0 = pltpu.async_copy(h0_hbm.at[fidx], i0, sem)
    c1 = pltpu.async_copy(h1_hbm.at[fidx], i1, sem)
    c0.wait()
    c1.wait()

    def gfill(k, _):
        sl = pl.ds(k * L, L)
        i0[sl] = i0[sl] * n_chunks + pat
        i1[sl] = i1[sl] * n_chunks + pat
        return _
    lax.fori_loop(0, KV, gfill, None)

    c0 = pltpu.async_copy(t0_hbm.at[i0], r0, sem)
    c1 = pltpu.async_copy(t1_hbm.at[i1], r1, sem)
    c0.wait()
    c1.wait()

    def addf(j, _):
        r0[j] = r0[j] + r1[j]
        return _
    lax.fori_loop(0, P, addf, None)

    pltpu.sync_copy(r0, out_hbm.at[pl.ds(wid * P, P)])


def kernel(x, table0, table1, h0, h1):
    rows, n_chunks, chunk_size = table0.shape
    B = x.shape[0]
    b_per_w = B // NW
    P = b_per_w * n_chunks

    t0f = table0.reshape(rows * n_chunks, chunk_size)
    t1f = table1.reshape(rows * n_chunks, chunk_size)
    h0f = h0.astype(jnp.int32).reshape(-1)
    h1f = h1.astype(jnp.int32).reshape(-1)
    xi = x.astype(jnp.int32)

    pos = jnp.arange(P, dtype=jnp.int32) // n_chunks

    mesh = plsc.VectorSubcoreMesh(core_axis_name="c", subcore_axis_name="s",
                                  num_cores=NC, num_subcores=NS)
    body = functools.partial(_body, b_per_w=b_per_w, n_chunks=n_chunks)
    out = pl.kernel(
        body,
        out_type=jax.ShapeDtypeStruct((B * n_chunks, chunk_size), jnp.float32),
        mesh=mesh,
        compiler_params=pltpu.CompilerParams(use_tc_tiling_on_sc=False,
                                             needs_layout_passes=False),
        scratch_types=[
            pltpu.VMEM((b_per_w,), jnp.int32),
            pltpu.VMEM((P,), jnp.int32),
            pltpu.VMEM((P,), jnp.int32),
            pltpu.VMEM((P,), jnp.int32),
            pltpu.VMEM((P,), jnp.int32),
            pltpu.VMEM((P, chunk_size), jnp.float32),
            pltpu.VMEM((P, chunk_size), jnp.float32),
            pltpu.SemaphoreType.DMA,
        ],
    )(xi, pos, t0f, t1f, h0f, h1f)
    return out.reshape(B, n_chunks * chunk_size)


# speedup vs baseline: 9.9022x; 9.9022x over previous
import functools

import jax
import jax.numpy as jnp
from jax import lax
from jax.experimental import pallas as pl
from jax.experimental.pallas import tpu as pltpu
from jax.experimental.pallas import tpu_sc as plsc

NC, NS, L = 2, 16, 16
NW = NC * NS


def _worker_id():
    return lax.axis_index("s") * NC + lax.axis_index("c")


def _hash_body(x_hbm, pos_hbm, h0m_hbm, h1m_hbm, h0t_hbm, h1t_hbm,
               g0_hbm, g1_hbm, xv, posv, xe, fidx, i0, i1, tl0, tl1, sem,
               *, b_per_w, n_chunks, vcut):
    wid = _worker_id()
    base = wid * b_per_w
    P = b_per_w * n_chunks
    KV = P // L

    pltpu.sync_copy(x_hbm.at[pl.ds(base, b_per_w)], xv)
    pltpu.sync_copy(pos_hbm, posv)
    pltpu.sync_copy(h0t_hbm, tl0)
    pltpu.sync_copy(h1t_hbm, tl1)

    lane = lax.iota(jnp.int32, L)
    pat = lane % n_chunks

    def fill(k, _):
        sl = pl.ds(k * L, L)
        xi = plsc.load_gather(xv, [posv[sl]])
        xe[sl] = xi
        xc = jnp.minimum(xi, vcut - 1)
        fidx[sl] = ((xc >> 7) << 9) | (pat << 7) | (xc & 127)
        return _
    lax.fori_loop(0, KV, fill, None)

    c0 = pltpu.async_copy(h0m_hbm.at[fidx], i0, sem)
    c1 = pltpu.async_copy(h1m_hbm.at[fidx], i1, sem)
    c0.wait()
    c1.wait()

    def gfill(k, _):
        sl = pl.ds(k * L, L)
        xev = xe[sl]
        tidx = jnp.maximum(xev - vcut, 0) * n_chunks + pat
        tv0 = plsc.load_gather(tl0, [tidx])
        tv1 = plsc.load_gather(tl1, [tidx])
        m = xev >= vcut
        iv0 = jnp.where(m, tv0, i0[sl])
        iv1 = jnp.where(m, tv1, i1[sl])
        i0[sl] = iv0 * n_chunks + pat
        i1[sl] = iv1 * n_chunks + pat
        return _
    lax.fori_loop(0, KV, gfill, None)

    pltpu.sync_copy(i0, g0_hbm.at[pl.ds(wid * P, P)])
    pltpu.sync_copy(i1, g1_hbm.at[pl.ds(wid * P, P)])


def _table_body(g0_hbm, g1_hbm, t0_hbm, t1_hbm, out_hbm,
                g0v, g1v, r0, r1, sem, *, b_per_w, n_chunks):
    wid = _worker_id()
    P = b_per_w * n_chunks

    pltpu.sync_copy(g0_hbm.at[pl.ds(wid * P, P)], g0v)
    pltpu.sync_copy(g1_hbm.at[pl.ds(wid * P, P)], g1v)

    c0 = pltpu.async_copy(t0_hbm.at[g0v], r0, sem)
    c1 = pltpu.async_copy(t1_hbm.at[g1v], r1, sem)
    c0.wait()
    c1.wait()

    def addf(j, _):
        j4 = j * 4
        r0[j4] = r0[j4] + r1[j4]
        r0[j4 + 1] = r0[j4 + 1] + r1[j4 + 1]
        r0[j4 + 2] = r0[j4 + 2] + r1[j4 + 2]
        r0[j4 + 3] = r0[j4 + 3] + r1[j4 + 3]
        return _
    lax.fori_loop(0, P // 4, addf, None)

    pltpu.sync_copy(r0, out_hbm.at[pl.ds(wid * P, P)])


def kernel(x, table0, table1, h0, h1):
    rows, n_chunks, chunk_size = table0.shape
    vocab = h0.shape[0]
    B = x.shape[0]
    b_per_w = B // NW
    P = b_per_w * n_chunks
    vcut = (vocab // 128) * 128

    t0f = table0.reshape(rows * n_chunks, chunk_size)
    t1f = table1.reshape(rows * n_chunks, chunk_size)
    h0m = h0[:vcut].reshape(vcut // 128, 128, n_chunks)
    h0m = h0m.transpose(0, 2, 1).reshape(-1)
    h1m = h1[:vcut].reshape(vcut // 128, 128, n_chunks)
    h1m = h1m.transpose(0, 2, 1).reshape(-1)
    h0t = h0[vcut:].reshape(-1)
    h1t = h1[vcut:].reshape(-1)
    ntail = vocab - vcut
    pos = jnp.arange(P, dtype=jnp.int32) // n_chunks

    mesh = plsc.VectorSubcoreMesh(core_axis_name="c", subcore_axis_name="s",
                                  num_cores=NC, num_subcores=NS)
    params = pltpu.CompilerParams(use_tc_tiling_on_sc=False,
                                  needs_layout_passes=False)

    hash_body = functools.partial(_hash_body, b_per_w=b_per_w,
                                  n_chunks=n_chunks, vcut=vcut)
    g0, g1 = pl.kernel(
        hash_body,
        out_type=(jax.ShapeDtypeStruct((B * n_chunks,), jnp.int32),
                  jax.ShapeDtypeStruct((B * n_chunks,), jnp.int32)),
        mesh=mesh,
        compiler_params=params,
        scratch_types=[
            pltpu.VMEM((b_per_w,), jnp.int32),
            pltpu.VMEM((P,), jnp.int32),
            pltpu.VMEM((P,), jnp.int32),
            pltpu.VMEM((P,), jnp.int32),
            pltpu.VMEM((P,), jnp.int32),
            pltpu.VMEM((P,), jnp.int32),
            pltpu.VMEM((ntail * n_chunks,), jnp.int32),
            pltpu.VMEM((ntail * n_chunks,), jnp.int32),
            pltpu.SemaphoreType.DMA,
        ],
    )(x, pos, h0m, h1m, h0t, h1t)

    table_body = functools.partial(_table_body, b_per_w=b_per_w,
                                   n_chunks=n_chunks)
    out = pl.kernel(
        table_body,
        out_type=jax.ShapeDtypeStruct((B * n_chunks, chunk_size), jnp.float32),
        mesh=mesh,
        compiler_params=params,
        scratch_types=[
            pltpu.VMEM((P,), jnp.int32),
            pltpu.VMEM((P,), jnp.int32),
            pltpu.VMEM((P, chunk_size), jnp.float32),
            pltpu.VMEM((P, chunk_size), jnp.float32),
            pltpu.SemaphoreType.DMA,
        ],
    )(g0, g1, t0f, t1f)
    return out.reshape(B, n_chunks * chunk_size)


# speedup vs baseline: 10.3898x; 1.0492x over previous
import functools

import jax
import jax.numpy as jnp
from jax import lax
from jax.experimental import pallas as pl
from jax.experimental.pallas import tpu as pltpu
from jax.experimental.pallas import tpu_sc as plsc

NC, NS, L = 2, 16, 16
NW = NC * NS


def _worker_id():
    return lax.axis_index("s") * NC + lax.axis_index("c")


def _body(x_hbm, pos_hbm, t0_hbm, t1_hbm, h0m_hbm, h1m_hbm, h0t_hbm, h1t_hbm,
          out_hbm, xv, posv, xe, fidx, i0, i1, tl0, tl1, r0, r1, sem,
          *, b_per_w, n_chunks, vcut):
    wid = _worker_id()
    base = wid * b_per_w
    P = b_per_w * n_chunks
    KV = P // L

    pltpu.sync_copy(x_hbm.at[pl.ds(base, b_per_w)], xv)
    pltpu.sync_copy(pos_hbm, posv)
    pltpu.sync_copy(h0t_hbm, tl0)
    pltpu.sync_copy(h1t_hbm, tl1)

    lane = lax.iota(jnp.int32, L)
    pat = lane % n_chunks

    def fill(k, _):
        sl = pl.ds(k * L, L)
        xi = plsc.load_gather(xv, [posv[sl]])
        xe[sl] = xi
        xc = jnp.minimum(xi, vcut - 1)
        fidx[sl] = pat * vcut + xc
        return _
    lax.fori_loop(0, KV, fill, None)

    c0 = pltpu.async_copy(h0m_hbm.at[fidx], i0, sem)
    c1 = pltpu.async_copy(h1m_hbm.at[fidx], i1, sem)
    c0.wait()
    c1.wait()

    def gfill(k, _):
        sl = pl.ds(k * L, L)
        xev = xe[sl]
        tidx = jnp.maximum(xev - vcut, 0) * n_chunks + pat
        tv0 = plsc.load_gather(tl0, [tidx])
        tv1 = plsc.load_gather(tl1, [tidx])
        m = xev >= vcut
        iv0 = jnp.where(m, tv0, i0[sl])
        iv1 = jnp.where(m, tv1, i1[sl])
        i0[sl] = iv0 * n_chunks + pat
        i1[sl] = iv1 * n_chunks + pat
        return _
    lax.fori_loop(0, KV, gfill, None)

    c0 = pltpu.async_copy(t0_hbm.at[i0], r0, sem)
    c1 = pltpu.async_copy(t1_hbm.at[i1], r1, sem)
    c0.wait()
    c1.wait()

    def addf(j, _):
        r0[j] = r0[j] + r1[j]
        return _
    lax.fori_loop(0, P, addf, None)

    pltpu.sync_copy(r0, out_hbm.at[pl.ds(wid * P, P)])


def kernel(x, table0, table1, h0, h1):
    rows, n_chunks, chunk_size = table0.shape
    vocab = h0.shape[0]
    B = x.shape[0]
    b_per_w = B // NW
    P = b_per_w * n_chunks
    vcut = (vocab // 128) * 128

    t0f = table0.reshape(rows * n_chunks, chunk_size)
    t1f = table1.reshape(rows * n_chunks, chunk_size)
    h0m = h0[:vcut].T.reshape(-1)
    h1m = h1[:vcut].T.reshape(-1)
    h0t = h0[vcut:].reshape(-1)
    h1t = h1[vcut:].reshape(-1)
    ntail = vocab - vcut
    pos = jnp.arange(P, dtype=jnp.int32) // n_chunks

    mesh = plsc.VectorSubcoreMesh(core_axis_name="c", subcore_axis_name="s",
                                  num_cores=NC, num_subcores=NS)
    body = functools.partial(_body, b_per_w=b_per_w, n_chunks=n_chunks,
                             vcut=vcut)
    out = pl.kernel(
        body,
        out_type=jax.ShapeDtypeStruct((B * n_chunks, chunk_size), jnp.float32),
        mesh=mesh,
        compiler_params=pltpu.CompilerParams(use_tc_tiling_on_sc=False,
                                             needs_layout_passes=False),
        scratch_types=[
            pltpu.VMEM((b_per_w,), jnp.int32),
            pltpu.VMEM((P,), jnp.int32),
            pltpu.VMEM((P,), jnp.int32),
            pltpu.VMEM((P,), jnp.int32),
            pltpu.VMEM((P,), jnp.int32),
            pltpu.VMEM((P,), jnp.int32),
            pltpu.VMEM((ntail * n_chunks,), jnp.int32),
            pltpu.VMEM((ntail * n_chunks,), jnp.int32),
            pltpu.VMEM((P, chunk_size), jnp.float32),
            pltpu.VMEM((P, chunk_size), jnp.float32),
            pltpu.SemaphoreType.DMA,
        ],
    )(x, pos, t0f, t1f, h0m, h1m, h0t, h1t)
    return out.reshape(B, n_chunks * chunk_size)


# speedup vs baseline: 12.7174x; 1.2240x over previous
import functools

import jax
import jax.numpy as jnp
from jax import lax
from jax.experimental import pallas as pl
from jax.experimental.pallas import tpu as pltpu
from jax.experimental.pallas import tpu_sc as plsc

NC, NS, L = 2, 16, 16
NW = NC * NS


def _worker_id():
    return lax.axis_index("s") * NC + lax.axis_index("c")


def _body(x_hbm, pos_hbm, t0_hbm, t1_hbm, hm_hbm, ht_hbm,
          out_hbm, xv, posv, xe, fidx, i0, i1, tlv, r0, r1, sem,
          *, b_per_w, n_chunks, vcut):
    wid = _worker_id()
    base = wid * b_per_w
    P = b_per_w * n_chunks
    KV = P // L

    pltpu.sync_copy(x_hbm.at[pl.ds(base, b_per_w)], xv)
    pltpu.sync_copy(pos_hbm, posv)
    pltpu.sync_copy(ht_hbm, tlv)

    lane = lax.iota(jnp.int32, L)
    pat = lane % n_chunks

    def fill(k, _):
        sl = pl.ds(k * L, L)
        xi = plsc.load_gather(xv, [posv[sl]])
        xe[sl] = xi
        xc = jnp.minimum(xi, vcut - 1)
        fidx[sl] = pat * vcut + xc
        return _
    lax.fori_loop(0, KV, fill, None)

    pltpu.async_copy(hm_hbm.at[fidx], i0, sem).wait()

    def gfill(k, _):
        sl = pl.ds(k * L, L)
        xev = xe[sl]
        tidx = jnp.maximum(xev - vcut, 0) * n_chunks + pat
        tv = plsc.load_gather(tlv, [tidx])
        m = xev >= vcut
        w = jnp.where(m, tv, i0[sl])
        i0[sl] = (w & 0xFFFF) * n_chunks + pat
        i1[sl] = ((w >> 16) & 0xFFFF) * n_chunks + pat
        return _
    lax.fori_loop(0, KV, gfill, None)

    c0 = pltpu.async_copy(t0_hbm.at[i0], r0, sem)
    c1 = pltpu.async_copy(t1_hbm.at[i1], r1, sem)
    c0.wait()
    c1.wait()

    def addf(j, _):
        r0[j] = r0[j] + r1[j]
        return _
    lax.fori_loop(0, P, addf, None)

    pltpu.sync_copy(r0, out_hbm.at[pl.ds(wid * P, P)])


def kernel(x, table0, table1, h0, h1):
    rows, n_chunks, chunk_size = table0.shape
    vocab = h0.shape[0]
    B = x.shape[0]
    b_per_w = B // NW
    P = b_per_w * n_chunks
    vcut = (vocab // 128) * 128

    t0f = table0.reshape(rows * n_chunks, chunk_size)
    t1f = table1.reshape(rows * n_chunks, chunk_size)
    hm = (h0[:vcut] | (h1[:vcut] << 16)).T.reshape(-1)
    ht = (h0[vcut:] | (h1[vcut:] << 16)).reshape(-1)
    ntail = vocab - vcut
    pos = jnp.arange(P, dtype=jnp.int32) // n_chunks

    mesh = plsc.VectorSubcoreMesh(core_axis_name="c", subcore_axis_name="s",
                                  num_cores=NC, num_subcores=NS)
    body = functools.partial(_body, b_per_w=b_per_w, n_chunks=n_chunks,
                             vcut=vcut)
    out = pl.kernel(
        body,
        out_type=jax.ShapeDtypeStruct((B * n_chunks, chunk_size), jnp.float32),
        mesh=mesh,
        compiler_params=pltpu.CompilerParams(use_tc_tiling_on_sc=False,
                                             needs_layout_passes=False),
        scratch_types=[
            pltpu.VMEM((b_per_w,), jnp.int32),
            pltpu.VMEM((P,), jnp.int32),
            pltpu.VMEM((P,), jnp.int32),
            pltpu.VMEM((P,), jnp.int32),
            pltpu.VMEM((P,), jnp.int32),
            pltpu.VMEM((P,), jnp.int32),
            pltpu.VMEM((ntail * n_chunks,), jnp.int32),
            pltpu.VMEM((P, chunk_size), jnp.float32),
            pltpu.VMEM((P, chunk_size), jnp.float32),
            pltpu.SemaphoreType.DMA,
        ],
    )(x, pos, t0f, t1f, hm, ht)
    return out.reshape(B, n_chunks * chunk_size)


# speedup vs baseline: 13.1874x; 1.0370x over previous
import functools

import jax
import jax.numpy as jnp
from jax import lax
from jax.experimental import pallas as pl
from jax.experimental.pallas import tpu as pltpu
from jax.experimental.pallas import tpu_sc as plsc

NC, NS, L = 2, 16, 16
NW = NC * NS


def _worker_id():
    return lax.axis_index("s") * NC + lax.axis_index("c")


def _body(x_hbm, pos_hbm, t0_hbm, t1_hbm, hm_hbm, ht_hbm,
          out_hbm, xv, posv, xe, fidx, i0, i1, tlv, r0, r1, sem,
          *, b_per_w, n_chunks, vcut):
    wid = _worker_id()
    base = wid * b_per_w
    P = b_per_w * n_chunks
    KV = P // L

    pltpu.sync_copy(x_hbm.at[pl.ds(base, b_per_w)], xv)
    pltpu.sync_copy(pos_hbm, posv)
    pltpu.sync_copy(ht_hbm, tlv)

    lane = lax.iota(jnp.int32, L)
    pat = lane % n_chunks

    def fill(k, _):
        sl = pl.ds(k * L, L)
        xi = plsc.load_gather(xv, [posv[sl]])
        xe[sl] = xi
        xc = jnp.minimum(xi, vcut - 1)
        fidx[sl] = pat * vcut + xc
        return _
    lax.fori_loop(0, KV, fill, None)

    pltpu.async_copy(hm_hbm.at[fidx], i0, sem).wait()

    def gfill(k, _):
        sl = pl.ds(k * L, L)
        xev = xe[sl]
        tidx = jnp.maximum(xev - vcut, 0) * n_chunks + pat
        tv = plsc.load_gather(tlv, [tidx])
        m = xev >= vcut
        w = jnp.where(m, tv, i0[sl])
        i0[sl] = (w & 0xFFFF) * n_chunks + pat
        i1[sl] = ((w >> 16) & 0xFFFF) * n_chunks + pat
        return _
    lax.fori_loop(0, KV, gfill, None)

    c0 = pltpu.async_copy(t0_hbm.at[i0], r0, sem)
    c1 = pltpu.async_copy(t1_hbm.at[i1], r1, sem)
    c0.wait()
    c1.wait()

    def addf(j, _):
        j4 = j * 4
        r0[j4] = r0[j4] + r1[j4]
        r0[j4 + 1] = r0[j4 + 1] + r1[j4 + 1]
        r0[j4 + 2] = r0[j4 + 2] + r1[j4 + 2]
        r0[j4 + 3] = r0[j4 + 3] + r1[j4 + 3]
        return _
    lax.fori_loop(0, P // 4, addf, None)

    pltpu.sync_copy(r0, out_hbm.at[pl.ds(wid * P, P)])


def kernel(x, table0, table1, h0, h1):
    rows, n_chunks, chunk_size = table0.shape
    vocab = h0.shape[0]
    B = x.shape[0]
    b_per_w = B // NW
    P = b_per_w * n_chunks
    vcut = (vocab // 128) * 128

    t0f = table0.reshape(rows * n_chunks, chunk_size)
    t1f = table1.reshape(rows * n_chunks, chunk_size)
    hm = (h0[:vcut] | (h1[:vcut] << 16)).T.reshape(-1)
    ht = (h0[vcut:] | (h1[vcut:] << 16)).reshape(-1)
    ntail = vocab - vcut
    pos = jnp.arange(P, dtype=jnp.int32) // n_chunks

    mesh = plsc.VectorSubcoreMesh(core_axis_name="c", subcore_axis_name="s",
                                  num_cores=NC, num_subcores=NS)
    body = functools.partial(_body, b_per_w=b_per_w, n_chunks=n_chunks,
                             vcut=vcut)
    out = pl.kernel(
        body,
        out_type=jax.ShapeDtypeStruct((B * n_chunks, chunk_size), jnp.float32),
        mesh=mesh,
        compiler_params=pltpu.CompilerParams(use_tc_tiling_on_sc=False,
                                             needs_layout_passes=False),
        scratch_types=[
            pltpu.VMEM((b_per_w,), jnp.int32),
            pltpu.VMEM((P,), jnp.int32),
            pltpu.VMEM((P,), jnp.int32),
            pltpu.VMEM((P,), jnp.int32),
            pltpu.VMEM((P,), jnp.int32),
            pltpu.VMEM((P,), jnp.int32),
            pltpu.VMEM((ntail * n_chunks,), jnp.int32),
            pltpu.VMEM((P, chunk_size), jnp.float32),
            pltpu.VMEM((P, chunk_size), jnp.float32),
            pltpu.SemaphoreType.DMA,
        ],
    )(x, pos, t0f, t1f, hm, ht)
    return out.reshape(B, n_chunks * chunk_size)


# speedup vs baseline: 13.1926x; 1.0004x over previous
import functools

import jax
import jax.numpy as jnp
from jax import lax
from jax.experimental import pallas as pl
from jax.experimental.pallas import tpu as pltpu
from jax.experimental.pallas import tpu_sc as plsc

NC, NS, L = 2, 16, 16
NW = NC * NS


def _worker_id():
    return lax.axis_index("s") * NC + lax.axis_index("c")


def _body(x_hbm, pos_hbm, t0_hbm, t1_hbm, hm_hbm, ht_hbm,
          out_hbm, xv, posv, xe, fidx, i0, i1, tlv, r0, r1, sem,
          *, b_per_w, n_chunks, vcut):
    wid = _worker_id()
    base = wid * b_per_w
    P = b_per_w * n_chunks
    KV = P // L

    pltpu.sync_copy(x_hbm.at[pl.ds(base, b_per_w)], xv)
    pltpu.sync_copy(pos_hbm, posv)
    pltpu.sync_copy(ht_hbm, tlv)

    lane = lax.iota(jnp.int32, L)
    pat = lane % n_chunks

    def fill1(sl):
        xi = plsc.load_gather(xv, [posv[sl]])
        xe[sl] = xi
        xc = jnp.minimum(xi, vcut - 1)
        fidx[sl] = pat * vcut + xc

    def fill(k, _):
        fill1(pl.ds(k * 2 * L, L))
        fill1(pl.ds(k * 2 * L + L, L))
        return _
    lax.fori_loop(0, KV // 2, fill, None)

    pltpu.async_copy(hm_hbm.at[fidx], i0, sem).wait()

    def gfill1(sl):
        xev = xe[sl]
        tidx = jnp.maximum(xev - vcut, 0) * n_chunks + pat
        tv = plsc.load_gather(tlv, [tidx])
        m = xev >= vcut
        w = jnp.where(m, tv, i0[sl])
        i0[sl] = (w & 0xFFFF) * n_chunks + pat
        i1[sl] = ((w >> 16) & 0xFFFF) * n_chunks + pat

    def gfill(k, _):
        gfill1(pl.ds(k * 2 * L, L))
        gfill1(pl.ds(k * 2 * L + L, L))
        return _
    lax.fori_loop(0, KV // 2, gfill, None)

    c0 = pltpu.async_copy(t0_hbm.at[i0], r0, sem)
    c1 = pltpu.async_copy(t1_hbm.at[i1], r1, sem)
    c0.wait()
    c1.wait()

    def addf(j, _):
        j4 = j * 4
        r0[j4] = r0[j4] + r1[j4]
        r0[j4 + 1] = r0[j4 + 1] + r1[j4 + 1]
        r0[j4 + 2] = r0[j4 + 2] + r1[j4 + 2]
        r0[j4 + 3] = r0[j4 + 3] + r1[j4 + 3]
        return _
    lax.fori_loop(0, P // 4, addf, None)

    pltpu.sync_copy(r0, out_hbm.at[pl.ds(wid * P, P)])


def kernel(x, table0, table1, h0, h1):
    rows, n_chunks, chunk_size = table0.shape
    vocab = h0.shape[0]
    B = x.shape[0]
    b_per_w = B // NW
    P = b_per_w * n_chunks
    vcut = (vocab // 128) * 128

    t0f = table0.reshape(rows * n_chunks, chunk_size)
    t1f = table1.reshape(rows * n_chunks, chunk_size)
    hm = (h0[:vcut] | (h1[:vcut] << 16)).T.reshape(-1)
    ht = (h0[vcut:] | (h1[vcut:] << 16)).reshape(-1)
    ntail = vocab - vcut
    pos = jnp.arange(P, dtype=jnp.int32) // n_chunks

    mesh = plsc.VectorSubcoreMesh(core_axis_name="c", subcore_axis_name="s",
                                  num_cores=NC, num_subcores=NS)
    body = functools.partial(_body, b_per_w=b_per_w, n_chunks=n_chunks,
                             vcut=vcut)
    out = pl.kernel(
        body,
        out_type=jax.ShapeDtypeStruct((B * n_chunks, chunk_size), jnp.float32),
        mesh=mesh,
        compiler_params=pltpu.CompilerParams(use_tc_tiling_on_sc=False,
                                             needs_layout_passes=False),
        scratch_types=[
            pltpu.VMEM((b_per_w,), jnp.int32),
            pltpu.VMEM((P,), jnp.int32),
            pltpu.VMEM((P,), jnp.int32),
            pltpu.VMEM((P,), jnp.int32),
            pltpu.VMEM((P,), jnp.int32),
            pltpu.VMEM((P,), jnp.int32),
            pltpu.VMEM((ntail * n_chunks,), jnp.int32),
            pltpu.VMEM((P, chunk_size), jnp.float32),
            pltpu.VMEM((P, chunk_size), jnp.float32),
            pltpu.SemaphoreType.DMA,
        ],
    )(x, pos, t0f, t1f, hm, ht)
    return out.reshape(B, n_chunks * chunk_size)
